# baseline (speedup 1.0000x reference)
import functools
import math

import jax
import jax.numpy as jnp
from jax import lax
from jax.experimental import pallas as pl
from jax.experimental.pallas import tpu as pltpu
from jax.experimental.pallas import tpu_sc as plsc

VOCAB = 1000000
D = 32
L_SEQ = 200
B_ROWS = 16384
N_FLAT = B_ROWS * L_SEQ
NUM_CORES = 2
NUM_SUBCORES = 16
NW = NUM_CORES * NUM_SUBCORES
PER_W = N_FLAT // NW
NBUF = 4
CHUNK = 512
STEPS = PER_W // CHUNK
ITERS = STEPS // NBUF
SCALE = math.sqrt(float(D))

_mesh = plsc.VectorSubcoreMesh(
    core_axis_name="c", subcore_axis_name="s",
    num_cores=NUM_CORES, num_subcores=NUM_SUBCORES)


@functools.partial(
    pl.kernel,
    out_type=jax.ShapeDtypeStruct((L_SEQ, D, B_ROWS, 1), jnp.float32),
    mesh=_mesh,
    scratch_types=[
        [pltpu.VMEM((CHUNK,), jnp.int32) for _ in range(NBUF)],
        [pltpu.VMEM((CHUNK, D), jnp.float32) for _ in range(NBUF)],
        pltpu.VMEM((8, D), jnp.float32),
        [pltpu.SemaphoreType.DMA for _ in range(NBUF)],
        [pltpu.SemaphoreType.DMA for _ in range(NBUF)],
    ],
    compiler_params=pltpu.CompilerParams(use_tc_tiling_on_sc=False,
                                         needs_layout_passes=False),
)
def _emb_lookup(x_hbm, table_hbm, pos_hbm, out_hbm,
                idx_v, rows_v, psb, sg, so):
    wid = lax.axis_index("s") * NUM_CORES + lax.axis_index("c")
    base = wid * PER_W
    l_first = base // B_ROWS
    pltpu.sync_copy(pos_hbm.at[pl.ds(l_first, 8)], psb)

    def out_col(g, d):
        off = base + g * CHUNK
        return out_hbm.at[off // B_ROWS, d, pl.ds(off % B_ROWS, CHUNK), :]

    def fma_chunk(rows, l):
        dl = l - l_first
        pv0 = psb[dl, pl.ds(0, 16)]
        pv1 = psb[dl, pl.ds(16, 16)]

        def rloop(q, c):
            for u in range(4):
                r = q * 4 + u
                rows[r, pl.ds(0, 16)] = rows[r, pl.ds(0, 16)] * SCALE + pv0
                rows[r, pl.ds(16, 16)] = rows[r, pl.ds(16, 16)] * SCALE + pv1
            return c
        lax.fori_loop(0, CHUNK // 4, rloop, 0)

    for b in range(2):
        pltpu.sync_copy(x_hbm.at[pl.ds(base + b * CHUNK, CHUNK)], idx_v[b])
        pltpu.async_copy(table_hbm.at[idx_v[b]], rows_v[b], sg[b])

    def ring(i, carry):
        for s in range(NBUF):
            g = i * NBUF + s
            off = base + g * CHUNK
            pltpu.make_async_copy(table_hbm.at[idx_v[s]], rows_v[s],
                                  sg[s]).wait()
            fma_chunk(rows_v[s], off // B_ROWS)
            for d in range(D):
                pltpu.async_copy(rows_v[s].at[:, pl.ds(d, 1)], out_col(g, d), so[s])

            t = (s + 2) % NBUF
            gp = g + 2

            @pl.when(gp < STEPS)
            def _():
                @pl.when(gp >= NBUF)
                def _():
                    for d in range(D):
                        pltpu.make_async_copy(
                            rows_v[t].at[:, pl.ds(d, 1)], out_col(gp - NBUF, d),
                            so[t]).wait()
                pltpu.sync_copy(
                    x_hbm.at[pl.ds(base + gp * CHUNK, CHUNK)], idx_v[t])
                pltpu.async_copy(table_hbm.at[idx_v[t]], rows_v[t], sg[t])
        return carry

    lax.fori_loop(0, ITERS, ring, 0)

    for b in range(NBUF):
        g = STEPS - NBUF + b
        for d in range(D):
            pltpu.make_async_copy(rows_v[b].at[:, pl.ds(d, 1)],
                                  out_col(g, d), so[b]).wait()


def kernel(x, table, pos_encoding):
    x_lmaj = x.T.reshape(-1).astype(jnp.int32)
    out = _emb_lookup(x_lmaj, table, pos_encoding)
    return out.reshape(L_SEQ, D, B_ROWS).transpose(2, 0, 1)


# speedup vs baseline: 107.3835x; 107.3835x over previous
import functools
import math

import jax
import jax.numpy as jnp
from jax import lax
from jax.experimental import pallas as pl
from jax.experimental.pallas import tpu as pltpu
from jax.experimental.pallas import tpu_sc as plsc

VOCAB = 1000000
D = 32
L_SEQ = 200
B_ROWS = 16384
N_FLAT = B_ROWS * L_SEQ
NUM_CORES = 2
NUM_SUBCORES = 16
NW = NUM_CORES * NUM_SUBCORES
PER_W = N_FLAT // NW
NBUF = 4
CHUNK = 512
STEPS = PER_W // CHUNK
ITERS = STEPS // NBUF
SCALE = math.sqrt(float(D))

_mesh = plsc.VectorSubcoreMesh(
    core_axis_name="c", subcore_axis_name="s",
    num_cores=NUM_CORES, num_subcores=NUM_SUBCORES)


@functools.partial(
    pl.kernel,
    out_type=jax.ShapeDtypeStruct((N_FLAT, D), jnp.float32),
    mesh=_mesh,
    scratch_types=[
        [pltpu.VMEM((CHUNK,), jnp.int32) for _ in range(NBUF)],
        [pltpu.VMEM((CHUNK, D), jnp.float32) for _ in range(NBUF)],
        pltpu.VMEM((L_SEQ, D), jnp.float32),
        [pltpu.SemaphoreType.DMA for _ in range(NBUF)],
        [pltpu.SemaphoreType.DMA for _ in range(NBUF)],
    ],
    compiler_params=pltpu.CompilerParams(use_tc_tiling_on_sc=False),
)
def _emb_lookup(x_hbm, table_hbm, pos_hbm, out_hbm,
                idx_v, rows_v, pos_v, sg, so):
    wid = lax.axis_index("s") * NUM_CORES + lax.axis_index("c")
    base = wid * PER_W
    pltpu.sync_copy(pos_hbm.at[pl.ds(0, L_SEQ)], pos_v)

    def fma_chunk(rows, l):
        pv0 = pos_v[l, pl.ds(0, 16)]
        pv1 = pos_v[l, pl.ds(16, 16)]

        def rloop(q, c):
            for u in range(4):
                r = q * 4 + u
                rows[r, pl.ds(0, 16)] = rows[r, pl.ds(0, 16)] * SCALE + pv0
                rows[r, pl.ds(16, 16)] = rows[r, pl.ds(16, 16)] * SCALE + pv1
            return c
        lax.fori_loop(0, CHUNK // 4, rloop, 0)

    for b in range(2):
        pltpu.sync_copy(x_hbm.at[pl.ds(base + b * CHUNK, CHUNK)], idx_v[b])
        pltpu.async_copy(table_hbm.at[idx_v[b]], rows_v[b], sg[b])

    def ring(i, carry):
        for s in range(NBUF):
            g = i * NBUF + s
            off = base + g * CHUNK
            pltpu.make_async_copy(table_hbm.at[idx_v[s]], rows_v[s],
                                  sg[s]).wait()
            fma_chunk(rows_v[s], off // B_ROWS)
            pltpu.async_copy(rows_v[s], out_hbm.at[pl.ds(off, CHUNK)], so[s])

            t = (s + 2) % NBUF
            gp = g + 2
            offp = base + gp * CHUNK

            @pl.when(gp < STEPS)
            def _():
                @pl.when(gp >= NBUF)
                def _():
                    pltpu.make_async_copy(
                        rows_v[t],
                        out_hbm.at[pl.ds(offp - NBUF * CHUNK, CHUNK)],
                        so[t]).wait()
                pltpu.sync_copy(x_hbm.at[pl.ds(offp, CHUNK)], idx_v[t])
                pltpu.async_copy(table_hbm.at[idx_v[t]], rows_v[t], sg[t])
        return carry

    lax.fori_loop(0, ITERS, ring, 0)

    for b in range(NBUF):
        off = base + (STEPS - NBUF + b) * CHUNK
        pltpu.make_async_copy(rows_v[b], out_hbm.at[pl.ds(off, CHUNK)],
                              so[b]).wait()


def kernel(x, table, pos_encoding):
    x_lmaj = x.T.reshape(-1).astype(jnp.int32)
    out = _emb_lookup(x_lmaj, table, pos_encoding)
    return out.reshape(L_SEQ, B_ROWS, D).transpose(1, 0, 2)
